# Initial kernel scaffold; baseline (speedup 1.0000x reference)
import jax
import jax.numpy as jnp
from jax.experimental import pallas as pl


def kernel(x, edge_index, W1, b1, W2, b2):
    raise NotImplementedError("write your pallas kernel here")


# speedup vs baseline: 130.0667x; 130.0667x over previous
import functools

import jax
import jax.numpy as jnp
from jax import lax
from jax.experimental import pallas as pl
from jax.experimental.pallas import tpu as pltpu
from jax.experimental.pallas import tpu_sc as plsc

NC = 2
NS = 16
NW = NC * NS
LANES = 128
CHUNK_ROWS = 8
CHUNK = CHUNK_ROWS * LANES

_f32 = jnp.float32


def _edge_pass(src2d, dst2d, tables, zeros):
    nt = len(tables)
    na = max(nt, 1)
    rows = src2d.shape[0]
    npad = zeros.shape[0]
    rows_per_worker = rows // NW
    n_chunks = rows_per_worker // CHUNK_ROWS
    sl = npad // NS

    scratch = (
        [pltpu.VMEM((CHUNK_ROWS, LANES), jnp.int32)] * 2
        + [pltpu.VMEM_SHARED((npad,), _f32)] * na
        + [pltpu.VMEM_SHARED((npad,), _f32)] * nt
        + ([pltpu.VMEM((CHUNK_ROWS, LANES), _f32)] * nt
           if nt else [pltpu.VMEM((LANES,), _f32)])
        + [pltpu.SemaphoreType.DMA] * 2
    )

    @functools.partial(
        pl.kernel,
        mesh=plsc.VectorSubcoreMesh(core_axis_name="c", subcore_axis_name="s"),
        out_type=tuple(jax.ShapeDtypeStruct((NC, npad), _f32)
                       for _ in range(na)),
        scratch_types=scratch,
        compiler_params=pltpu.CompilerParams(use_tc_tiling_on_sc=False),
    )
    def run(*args):
        it = iter(args)
        src_h, dst_h = next(it), next(it)
        tab_h = [next(it) for _ in range(nt)]
        z_h = next(it)
        out_h = [next(it) for _ in range(na)]
        sidx, didx = next(it), next(it)
        acc = [next(it) for _ in range(na)]
        tab = [next(it) for _ in range(nt)]
        if nt:
            rbuf = [next(it) for _ in range(nt)]
        else:
            ones_v = next(it)
        sem_g, sem_s = next(it), next(it)

        cid = lax.axis_index("c")
        sid = lax.axis_index("s")
        wid = cid * NS + sid
        tsl = pl.ds(sid * sl, sl)

        for a in acc:
            pltpu.sync_copy(z_h.at[tsl], a.at[tsl])
        for k in range(nt):
            pltpu.sync_copy(tab_h[k].at[tsl], tab[k].at[tsl])
        if not nt:
            for k in range(LANES // 16):
                ones_v[pl.ds(k * 16, 16)] = jnp.ones((16,), _f32)
        plsc.subcore_barrier()

        row0 = wid * rows_per_worker

        def body(g, carry):
            base = row0 + g * CHUNK_ROWS
            pltpu.sync_copy(src_h.at[pl.ds(base, CHUNK_ROWS)], sidx)
            pltpu.sync_copy(dst_h.at[pl.ds(base, CHUNK_ROWS)], didx)
            if nt:
                cps = [pltpu.async_copy(tab[k].at[sidx.at[j]],
                                        rbuf[k].at[j], sem_g)
                       for k in range(nt) for j in range(CHUNK_ROWS)]
                for c in cps:
                    c.wait()
                cps = [pltpu.async_copy(rbuf[k].at[j],
                                        acc[k].at[didx.at[j]], sem_s, add=True)
                       for k in range(nt) for j in range(CHUNK_ROWS)]
            else:
                cps = [pltpu.async_copy(ones_v, acc[0].at[didx.at[j]],
                                        sem_s, add=True)
                       for j in range(CHUNK_ROWS)]
            for c in cps:
                c.wait()
            return carry

        lax.fori_loop(0, n_chunks, body, 0)

        plsc.subcore_barrier()
        for k in range(na):
            pltpu.sync_copy(acc[k].at[tsl], out_h[k].at[cid, tsl])

    return run(src2d, dst2d, *tables, zeros)


def _tc1_body(degp_ref, xt_ref, dinv_ref, yt_ref):
    deg = degp_ref[0] + degp_ref[1] + 1.0
    dinv = lax.rsqrt(deg)
    dinv_ref[...] = dinv
    for c in range(3):
        yt_ref[c] = xt_ref[c] * dinv


def _tc2_body(s1a_ref, s1b_ref, s1c_ref, yt_ref, dinv_ref, w1_ref, b1_ref,
              w2_ref, z_ref):
    dinv = dinv_ref[...]
    s1 = [s1a_ref, s1b_ref, s1c_ref]
    t = [dinv * (s1[c][0] + s1[c][1] + yt_ref[c]) for c in range(3)]
    z = jnp.zeros_like(dinv)
    for j in range(16):
        h = (b1_ref[0, j] + t[0] * w1_ref[0, j] + t[1] * w1_ref[1, j]
             + t[2] * w1_ref[2, j])
        z = z + jnp.maximum(h, 0.0) * w2_ref[j, 0]
    z_ref[...] = z * dinv


def _tc3_body(s2p_ref, z_ref, dinv_ref, b2_ref, out_ref):
    t = dinv_ref[...] * (s2p_ref[0] + s2p_ref[1] + z_ref[...]) + b2_ref[0, 0]
    out_ref[...] = jax.nn.sigmoid(t)


def _vspec():
    return pl.BlockSpec(memory_space=pltpu.VMEM)


def _sspec():
    return pl.BlockSpec(memory_space=pltpu.SMEM)


def kernel(x, edge_index, W1, b1, W2, b2):
    n = x.shape[0]
    e = edge_index.shape[1]
    npad = -(-n // 2048) * 2048
    r128 = npad // LANES
    e_pad = -(-e // (NW * CHUNK)) * (NW * CHUNK)

    src = edge_index[0].astype(jnp.int32)
    dst = edge_index[1].astype(jnp.int32)
    pad_cnt = e_pad - e
    if pad_cnt:
        pad_idx = n + (jnp.arange(pad_cnt, dtype=jnp.int32) % (npad - n))
        src = jnp.concatenate([src, pad_idx])
        dst = jnp.concatenate([dst, pad_idx])
    src2d = src.reshape(e_pad // LANES, LANES)
    dst2d = dst.reshape(e_pad // LANES, LANES)

    zeros1 = jnp.zeros((npad,), _f32)

    (degp,) = _edge_pass(src2d, dst2d, (), zeros1)

    xp = jnp.pad(x, ((0, npad - n), (0, 0)))
    xt = xp.T.reshape(3, r128, LANES)
    dinv, yt = pl.pallas_call(
        _tc1_body,
        out_shape=(jax.ShapeDtypeStruct((r128, LANES), _f32),
                   jax.ShapeDtypeStruct((3, r128, LANES), _f32)),
        in_specs=[_vspec(), _vspec()],
        out_specs=(_vspec(), _vspec()),
    )(degp.reshape(2, r128, LANES), xt)

    ycols = tuple(yt[c].reshape(npad) for c in range(3))
    s1 = _edge_pass(src2d, dst2d, ycols, zeros1)

    s1r = [s.reshape(2, r128, LANES) for s in s1]
    z = pl.pallas_call(
        _tc2_body,
        out_shape=jax.ShapeDtypeStruct((r128, LANES), _f32),
        in_specs=[_vspec(), _vspec(), _vspec(), _vspec(), _vspec(),
                  _sspec(), _sspec(), _sspec()],
        out_specs=_vspec(),
    )(s1r[0], s1r[1], s1r[2], yt, dinv, W1, b1.reshape(1, 16), W2)

    (s2,) = _edge_pass(src2d, dst2d, (z.reshape(npad),), zeros1)

    out = pl.pallas_call(
        _tc3_body,
        out_shape=jax.ShapeDtypeStruct((r128, LANES), _f32),
        in_specs=[_vspec(), _vspec(), _vspec(), _sspec()],
        out_specs=_vspec(),
    )(s2.reshape(2, r128, LANES), z, dinv, b2.reshape(1, 1))

    return out.reshape(npad, 1)[:n]


# speedup vs baseline: 209.9036x; 1.6138x over previous
import functools

import jax
import jax.numpy as jnp
from jax import lax
from jax.experimental import pallas as pl
from jax.experimental.pallas import tpu as pltpu
from jax.experimental.pallas import tpu_sc as plsc

NC = 2
NS = 16
NW = NC * NS
LANES = 128
CHUNK = 4096

_f32 = jnp.float32


def _edge_pass(src1d, dst1d, tables, zeros):
    nt = len(tables)
    na = max(nt, 1)
    e_tot = src1d.shape[0]
    npad = zeros.shape[0]
    edges_per_worker = e_tot // NW
    n_chunks = edges_per_worker // CHUNK
    sl = npad // NS

    scratch = (
        [pltpu.VMEM((CHUNK,), jnp.int32)] * 2
        + [pltpu.VMEM_SHARED((npad,), _f32)] * na
        + [pltpu.VMEM_SHARED((npad,), _f32)] * nt
        + [pltpu.VMEM((CHUNK,), _f32)] * max(nt, 1)
        + [pltpu.SemaphoreType.DMA] * 2
    )

    @functools.partial(
        pl.kernel,
        mesh=plsc.VectorSubcoreMesh(core_axis_name="c", subcore_axis_name="s"),
        out_type=tuple(jax.ShapeDtypeStruct((NC, npad), _f32)
                       for _ in range(na)),
        scratch_types=scratch,
        compiler_params=pltpu.CompilerParams(use_tc_tiling_on_sc=False),
    )
    def run(*args):
        it = iter(args)
        src_h, dst_h = next(it), next(it)
        tab_h = [next(it) for _ in range(nt)]
        z_h = next(it)
        out_h = [next(it) for _ in range(na)]
        sidx, didx = next(it), next(it)
        acc = [next(it) for _ in range(na)]
        tab = [next(it) for _ in range(nt)]
        rbuf = [next(it) for _ in range(max(nt, 1))]
        sem_g, sem_s = next(it), next(it)

        cid = lax.axis_index("c")
        sid = lax.axis_index("s")
        wid = cid * NS + sid
        tsl = pl.ds(sid * sl, sl)

        for a in acc:
            pltpu.sync_copy(z_h.at[tsl], a.at[tsl])
        for k in range(nt):
            pltpu.sync_copy(tab_h[k].at[tsl], tab[k].at[tsl])
        if not nt:
            for k in range(CHUNK // 16):
                rbuf[0][pl.ds(k * 16, 16)] = jnp.ones((16,), _f32)
        plsc.subcore_barrier()

        e0 = wid * edges_per_worker

        def body(g, carry):
            base = e0 + g * CHUNK
            pltpu.sync_copy(src_h.at[pl.ds(base, CHUNK)], sidx)
            pltpu.sync_copy(dst_h.at[pl.ds(base, CHUNK)], didx)
            if nt:
                cps = [pltpu.async_copy(tab[k].at[sidx], rbuf[k], sem_g)
                       for k in range(nt)]
                for c in cps:
                    c.wait()
            cps = [pltpu.async_copy(rbuf[k], acc[k].at[didx], sem_s, add=True)
                   for k in range(na)]
            for c in cps:
                c.wait()
            return carry

        lax.fori_loop(0, n_chunks, body, 0)

        plsc.subcore_barrier()
        for k in range(na):
            pltpu.sync_copy(acc[k].at[tsl], out_h[k].at[cid, tsl])

    return run(src1d, dst1d, *tables, zeros)


def _tc1_body(degp_ref, xt_ref, dinv_ref, yt_ref):
    deg = degp_ref[0] + degp_ref[1] + 1.0
    dinv = lax.rsqrt(deg)
    dinv_ref[...] = dinv
    for c in range(3):
        yt_ref[c] = xt_ref[c] * dinv


def _tc2_body(s1a_ref, s1b_ref, s1c_ref, yt_ref, dinv_ref, w1_ref, b1_ref,
              w2_ref, z_ref):
    dinv = dinv_ref[...]
    s1 = [s1a_ref, s1b_ref, s1c_ref]
    t = [dinv * (s1[c][0] + s1[c][1] + yt_ref[c]) for c in range(3)]
    z = jnp.zeros_like(dinv)
    for j in range(16):
        h = (b1_ref[0, j] + t[0] * w1_ref[0, j] + t[1] * w1_ref[1, j]
             + t[2] * w1_ref[2, j])
        z = z + jnp.maximum(h, 0.0) * w2_ref[j, 0]
    z_ref[...] = z * dinv


def _tc3_body(s2p_ref, z_ref, dinv_ref, b2_ref, out_ref):
    t = dinv_ref[...] * (s2p_ref[0] + s2p_ref[1] + z_ref[...]) + b2_ref[0, 0]
    out_ref[...] = jax.nn.sigmoid(t)


def _vspec():
    return pl.BlockSpec(memory_space=pltpu.VMEM)


def _sspec():
    return pl.BlockSpec(memory_space=pltpu.SMEM)


def kernel(x, edge_index, W1, b1, W2, b2):
    n = x.shape[0]
    e = edge_index.shape[1]
    npad = -(-n // 2048) * 2048
    r128 = npad // LANES
    e_pad = -(-e // (NW * CHUNK)) * (NW * CHUNK)

    src = edge_index[0].astype(jnp.int32)
    dst = edge_index[1].astype(jnp.int32)
    pad_cnt = e_pad - e
    if pad_cnt:
        pad_idx = n + (jnp.arange(pad_cnt, dtype=jnp.int32) % (npad - n))
        src = jnp.concatenate([src, pad_idx])
        dst = jnp.concatenate([dst, pad_idx])

    zeros1 = jnp.zeros((npad,), _f32)

    (degp,) = _edge_pass(src, dst, (), zeros1)

    xp = jnp.pad(x, ((0, npad - n), (0, 0)))
    xt = xp.T.reshape(3, r128, LANES)
    dinv, yt = pl.pallas_call(
        _tc1_body,
        out_shape=(jax.ShapeDtypeStruct((r128, LANES), _f32),
                   jax.ShapeDtypeStruct((3, r128, LANES), _f32)),
        in_specs=[_vspec(), _vspec()],
        out_specs=(_vspec(), _vspec()),
    )(degp.reshape(2, r128, LANES), xt)

    ycols = tuple(yt[c].reshape(npad) for c in range(3))
    s1 = _edge_pass(src, dst, ycols, zeros1)

    s1r = [s.reshape(2, r128, LANES) for s in s1]
    z = pl.pallas_call(
        _tc2_body,
        out_shape=jax.ShapeDtypeStruct((r128, LANES), _f32),
        in_specs=[_vspec(), _vspec(), _vspec(), _vspec(), _vspec(),
                  _sspec(), _sspec(), _sspec()],
        out_specs=_vspec(),
    )(s1r[0], s1r[1], s1r[2], yt, dinv, W1, b1.reshape(1, 16), W2)

    (s2,) = _edge_pass(src, dst, (z.reshape(npad),), zeros1)

    out = pl.pallas_call(
        _tc3_body,
        out_shape=jax.ShapeDtypeStruct((r128, LANES), _f32),
        in_specs=[_vspec(), _vspec(), _vspec(), _sspec()],
        out_specs=_vspec(),
    )(s2.reshape(2, r128, LANES), z, dinv, b2.reshape(1, 1))

    return out.reshape(npad, 1)[:n]


# speedup vs baseline: 255.9304x; 1.2193x over previous
import functools

import jax
import jax.numpy as jnp
from jax import lax
from jax.experimental import pallas as pl
from jax.experimental.pallas import tpu as pltpu
from jax.experimental.pallas import tpu_sc as plsc

NC = 2
NS = 16
NW = NC * NS
LANES = 128
TARGET_CHUNK = 4096

_f32 = jnp.float32


def _edge_plan(e):
    epw = -(-e // (NW * 16)) * 16
    n_chunks = max(2, -(-epw // TARGET_CHUNK))
    n_chunks += n_chunks % 2
    chunk = -(-epw // (n_chunks * 8)) * 8
    return NW * chunk * n_chunks, chunk, n_chunks


def _edge_pass(src1d, dst1d, tables, zeros, chunk, n_chunks):
    nt = len(tables)
    na = max(nt, 1)
    npad = zeros.shape[0]
    edges_per_worker = chunk * n_chunks
    sl = npad // NS
    n_idx = 2 if nt else 1

    scratch = (
        [pltpu.VMEM((chunk,), jnp.int32)] * (2 * n_idx)
        + [pltpu.VMEM_SHARED((npad,), _f32)] * na
        + [pltpu.VMEM_SHARED((npad,), _f32)] * nt
        + [pltpu.VMEM((chunk,), _f32)] * (2 * nt if nt else 1)
        + [pltpu.SemaphoreType.DMA] * 4
    )

    @functools.partial(
        pl.kernel,
        mesh=plsc.VectorSubcoreMesh(core_axis_name="c", subcore_axis_name="s"),
        out_type=tuple(jax.ShapeDtypeStruct((NC, npad), _f32)
                       for _ in range(na)),
        scratch_types=scratch,
        compiler_params=pltpu.CompilerParams(use_tc_tiling_on_sc=False),
    )
    def run(*args):
        it = iter(args)
        src_h, dst_h = next(it), next(it)
        tab_h = [next(it) for _ in range(nt)]
        z_h = next(it)
        out_h = [next(it) for _ in range(na)]
        if nt:
            sidx = [next(it), next(it)]
        didx = [next(it), next(it)]
        acc = [next(it) for _ in range(na)]
        tab = [next(it) for _ in range(nt)]
        if nt:
            rbuf = [[next(it) for _ in range(nt)] for _ in range(2)]
        else:
            ones_v = next(it)
        sem_i, sem_g, sem_sa, sem_sb = next(it), next(it), next(it), next(it)

        cid = lax.axis_index("c")
        sid = lax.axis_index("s")
        wid = cid * NS + sid
        tsl = pl.ds(sid * sl, sl)

        for a in acc:
            pltpu.sync_copy(z_h.at[tsl], a.at[tsl])
        for k in range(nt):
            pltpu.sync_copy(tab_h[k].at[tsl], tab[k].at[tsl])
        if not nt:
            for k in range(chunk // 16):
                ones_v[pl.ds(k * 16, 16)] = jnp.ones((16,), _f32)
        plsc.subcore_barrier()

        e0 = wid * edges_per_worker

        def load_idx(base, p):
            cps = []
            if nt:
                cps.append(pltpu.async_copy(src_h.at[pl.ds(base, chunk)],
                                            sidx[p], sem_i))
            cps.append(pltpu.async_copy(dst_h.at[pl.ds(base, chunk)],
                                        didx[p], sem_i))
            return cps

        def gathers(p):
            return [pltpu.async_copy(tab[k].at[sidx[p]], rbuf[p][k], sem_g)
                    for k in range(nt)]

        def scatters(p, sem):
            srcs = rbuf[p] if nt else [ones_v]
            return [pltpu.async_copy(srcs[k], acc[k].at[didx[p]], sem,
                                     add=True)
                    for k in range(na)]

        def body(i, carry):
            base = e0 + i * (2 * chunk)
            ia = load_idx(base, 0)
            for c in ia:
                c.wait()
            ga = gathers(0)
            ib = load_idx(base + chunk, 1)
            for c in ga:
                c.wait()
            sa = scatters(0, sem_sa)
            for c in ib:
                c.wait()
            gb = gathers(1)
            for c in gb:
                c.wait()
            sb = scatters(1, sem_sb)
            for c in sa:
                c.wait()
            for c in sb:
                c.wait()
            return carry

        lax.fori_loop(0, n_chunks // 2, body, 0)

        plsc.subcore_barrier()
        for k in range(na):
            pltpu.sync_copy(acc[k].at[tsl], out_h[k].at[cid, tsl])

    return run(src1d, dst1d, *tables, zeros)


def _tc1_body(degp_ref, xt_ref, dinv_ref, yt_ref):
    deg = degp_ref[0] + degp_ref[1] + 1.0
    dinv = lax.rsqrt(deg)
    dinv_ref[...] = dinv
    for c in range(3):
        yt_ref[c] = xt_ref[c] * dinv


def _tc2_body(s1a_ref, s1b_ref, s1c_ref, yt_ref, dinv_ref, w1_ref, b1_ref,
              w2_ref, z_ref):
    dinv = dinv_ref[...]
    s1 = [s1a_ref, s1b_ref, s1c_ref]
    t = [dinv * (s1[c][0] + s1[c][1] + yt_ref[c]) for c in range(3)]
    z = jnp.zeros_like(dinv)
    for j in range(16):
        h = (b1_ref[0, j] + t[0] * w1_ref[0, j] + t[1] * w1_ref[1, j]
             + t[2] * w1_ref[2, j])
        z = z + jnp.maximum(h, 0.0) * w2_ref[j, 0]
    z_ref[...] = z * dinv


def _tc3_body(s2p_ref, z_ref, dinv_ref, b2_ref, out_ref):
    t = dinv_ref[...] * (s2p_ref[0] + s2p_ref[1] + z_ref[...]) + b2_ref[0, 0]
    out_ref[...] = jax.nn.sigmoid(t)


def _vspec():
    return pl.BlockSpec(memory_space=pltpu.VMEM)


def _sspec():
    return pl.BlockSpec(memory_space=pltpu.SMEM)


def kernel(x, edge_index, W1, b1, W2, b2):
    n = x.shape[0]
    e = edge_index.shape[1]
    npad = -(-n // 2048) * 2048
    r128 = npad // LANES
    e_pad, chunk, n_chunks = _edge_plan(e)

    src = edge_index[0].astype(jnp.int32)
    dst = edge_index[1].astype(jnp.int32)
    pad_cnt = e_pad - e
    if pad_cnt:
        pad_idx = n + (jnp.arange(pad_cnt, dtype=jnp.int32) % (npad - n))
        src = jnp.concatenate([src, pad_idx])
        dst = jnp.concatenate([dst, pad_idx])

    zeros1 = jnp.zeros((npad,), _f32)

    (degp,) = _edge_pass(src, dst, (), zeros1, chunk, n_chunks)

    xp = jnp.pad(x, ((0, npad - n), (0, 0)))
    xt = xp.T.reshape(3, r128, LANES)
    dinv, yt = pl.pallas_call(
        _tc1_body,
        out_shape=(jax.ShapeDtypeStruct((r128, LANES), _f32),
                   jax.ShapeDtypeStruct((3, r128, LANES), _f32)),
        in_specs=[_vspec(), _vspec()],
        out_specs=(_vspec(), _vspec()),
    )(degp.reshape(2, r128, LANES), xt)

    ycols = tuple(yt[c].reshape(npad) for c in range(3))
    s1 = _edge_pass(src, dst, ycols, zeros1, chunk, n_chunks)

    s1r = [s.reshape(2, r128, LANES) for s in s1]
    z = pl.pallas_call(
        _tc2_body,
        out_shape=jax.ShapeDtypeStruct((r128, LANES), _f32),
        in_specs=[_vspec(), _vspec(), _vspec(), _vspec(), _vspec(),
                  _sspec(), _sspec(), _sspec()],
        out_specs=_vspec(),
    )(s1r[0], s1r[1], s1r[2], yt, dinv, W1, b1.reshape(1, 16), W2)

    (s2,) = _edge_pass(src, dst, (z.reshape(npad),), zeros1,
                       chunk, n_chunks)

    out = pl.pallas_call(
        _tc3_body,
        out_shape=jax.ShapeDtypeStruct((r128, LANES), _f32),
        in_specs=[_vspec(), _vspec(), _vspec(), _sspec()],
        out_specs=_vspec(),
    )(s2.reshape(2, r128, LANES), z, dinv, b2.reshape(1, 1))

    return out.reshape(npad, 1)[:n]
